# Initial kernel scaffold; baseline (speedup 1.0000x reference)
import functools

import jax
import jax.numpy as jnp
from jax import lax
from jax.experimental import pallas as pl
from jax.experimental.pallas import tpu as pltpu
from jax.experimental.pallas import tpu_sc as plsc

L = 200
B = 4096
D = 32
CHUNK = 128
NC, NS = 2, 16
NW = NC * NS
NCHUNKS = (L * B) // CHUNK
CPW = NCHUNKS // NW
G = 8
NG = CPW // G

_mesh = plsc.VectorSubcoreMesh(
    core_axis_name="c", subcore_axis_name="s", num_cores=NC, num_subcores=NS
)


@functools.partial(
    pl.kernel,
    out_type=jax.ShapeDtypeStruct((NCHUNKS, CHUNK, D), jnp.float32),
    mesh=_mesh,
    scratch_types=[
        pltpu.VMEM((CPW, CHUNK), jnp.int32),
        pltpu.VMEM((2, G, CHUNK, D), jnp.float32),
        pltpu.SemaphoreType.DMA,
        pltpu.SemaphoreType.DMA,
    ],
)
def _embed_sc(table_hbm, idx_hbm, out_hbm, idx_v, rows_v, sem_g, sem_s):
    wid = lax.axis_index("s") * NC + lax.axis_index("c")
    c0 = wid * CPW

    pltpu.sync_copy(idx_hbm.at[pl.ds(c0, CPW)], idx_v)

    def issue_gathers(g, dbuf):
        for i in range(G):
            pltpu.async_copy(
                table_hbm.at[idx_v.at[g * G + i]], rows_v.at[dbuf, i], sem_g
            )

    def wait_gathers():
        for i in range(G):
            pltpu.make_async_copy(
                table_hbm.at[idx_v.at[0]], rows_v.at[0, i], sem_g
            ).wait()

    def issue_scatters(g, dbuf):
        for i in range(G):
            pltpu.async_copy(
                rows_v.at[dbuf, i], out_hbm.at[c0 + g * G + i], sem_s
            )

    def wait_scatters():
        for i in range(G):
            pltpu.make_async_copy(
                rows_v.at[0, i], out_hbm.at[0], sem_s
            ).wait()

    issue_gathers(0, 0)

    def body(g, _):
        dbuf = g % 2
        wait_gathers()
        issue_scatters(g, dbuf)

        @pl.when(g + 1 < NG)
        def _():
            @pl.when(g >= 1)
            def _():
                wait_scatters()

            issue_gathers(g + 1, 1 - dbuf)

        return 0

    lax.fori_loop(0, NG, body, 0)
    wait_scatters()


def kernel(x, table):
    idx = jnp.transpose(x).reshape(NCHUNKS, CHUNK)
    out = _embed_sc(table, idx)
    return out.reshape(L, B, D)


# speedup vs baseline: 5.7055x; 5.7055x over previous
import functools

import jax
import jax.numpy as jnp
from jax import lax
from jax.experimental import pallas as pl
from jax.experimental.pallas import tpu as pltpu
from jax.experimental.pallas import tpu_sc as plsc

L = 200
B = 4096
D = 32
DP = 128
CHUNK = 128
NC, NS = 2, 16
NW = NC * NS
NCHUNKS = (L * B) // CHUNK
CPW = NCHUNKS // NW
NBUF = 5
PG = 2
KS = 2

_mesh = plsc.VectorSubcoreMesh(
    core_axis_name="c", subcore_axis_name="s", num_cores=NC, num_subcores=NS
)


@functools.partial(
    pl.kernel,
    out_type=jax.ShapeDtypeStruct((NCHUNKS, CHUNK, D), jnp.float32),
    mesh=_mesh,
    scratch_types=[
        pltpu.VMEM((CPW, CHUNK), jnp.int32),
        pltpu.VMEM((NBUF, CHUNK, D), jnp.float32),
        pltpu.SemaphoreType.DMA,
        pltpu.SemaphoreType.DMA,
    ],
    compiler_params=pltpu.CompilerParams(use_tc_tiling_on_sc=False),
)
def _embed_sc(table_hbm, idx_hbm, out_hbm, idx_v, ring, sem_g, sem_s):
    wid = lax.axis_index("s") * NC + lax.axis_index("c")
    c0 = wid * CPW

    pltpu.sync_copy(idx_hbm.at[pl.ds(c0, CPW)], idx_v)

    def issue_gather(j):
        pltpu.async_copy(table_hbm.at[idx_v.at[j]], ring.at[j % NBUF], sem_g)

    def wait_gather():
        pltpu.make_async_copy(
            table_hbm.at[idx_v.at[0]], ring.at[0], sem_g
        ).wait()

    def issue_scatter(j):
        pltpu.async_copy(ring.at[j % NBUF], out_hbm.at[c0 + j], sem_s)

    def wait_scatter():
        pltpu.make_async_copy(ring.at[0], out_hbm.at[0], sem_s).wait()

    for j in range(PG + 1):
        issue_gather(j)

    def body(j, _):
        wait_gather()
        issue_scatter(j)

        @pl.when(j >= KS)
        def _():
            wait_scatter()

        @pl.when(j + PG + 1 < CPW)
        def _():
            issue_gather(j + PG + 1)

        return 0

    lax.fori_loop(0, CPW, body, 0)
    for _ in range(KS):
        wait_scatter()


def kernel(x, table):
    idx = jnp.transpose(x).reshape(NCHUNKS, CHUNK)
    out = _embed_sc(table, idx)
    return out.reshape(L, B, D)
